# baseline (speedup 1.0000x reference)
import functools

import jax
import jax.numpy as jnp
from jax.experimental import pallas as pl
from jax.experimental.pallas import tpu as pltpu

HIDDEN = 256
BM = 640


def _pad_rows(x, bm):
    pad = (-x.shape[0]) % bm
    if pad:
        x = jnp.pad(x, ((0, pad),) + ((0, 0),) * (x.ndim - 1))
    return x


def _lin_body(x_ref, w_ref, b_ref, o_ref):
    o_ref[:] = (
        jnp.dot(x_ref[:], w_ref[:], preferred_element_type=jnp.float32) + b_ref[:]
    )


def _lin_ln_silu_body(x_ref, w_ref, b_ref, g_ref, be_ref, o_ref):
    h = jnp.dot(x_ref[:], w_ref[:], preferred_element_type=jnp.float32) + b_ref[:]
    mu = jnp.mean(h, axis=-1, keepdims=True)
    var = jnp.mean((h - mu) ** 2, axis=-1, keepdims=True)
    h = (h - mu) * jax.lax.rsqrt(var + 1e-5) * g_ref[:] + be_ref[:]
    o_ref[:] = h * jax.nn.sigmoid(h)


def _linear(x, W, b, g=None, be=None):
    m0, k = x.shape
    f = W.shape[1]
    xp = _pad_rows(x, BM)
    m = xp.shape[0]
    args = [xp, W, b.reshape(1, f)]
    in_specs = [
        pl.BlockSpec((BM, k), lambda i: (i, 0)),
        pl.BlockSpec((k, f), lambda i: (0, 0)),
        pl.BlockSpec((1, f), lambda i: (0, 0)),
    ]
    if g is not None:
        args += [g.reshape(1, f), be.reshape(1, f)]
        in_specs += [
            pl.BlockSpec((1, f), lambda i: (0, 0)),
            pl.BlockSpec((1, f), lambda i: (0, 0)),
        ]
        body = _lin_ln_silu_body
    else:
        body = _lin_body
    out = pl.pallas_call(
        body,
        grid=(m // BM,),
        in_specs=in_specs,
        out_specs=pl.BlockSpec((BM, f), lambda i: (i, 0)),
        out_shape=jax.ShapeDtypeStruct((m, f), jnp.float32),
    )(*args)
    return out[:m0]


def _mlp(x, p):
    return _linear(x, p["W"], p["b"], p["g"], p["be"])


def _edge_body(xs_ref, xd_ref, ye_ref, bh_ref, y_ref, g_ref, be_ref,
               cat_ref, ynew_ref):
    m = xs_ref[:] + xd_ref[:] + ye_ref[:]
    s = jax.nn.sigmoid(m)
    cat_ref[:, :HIDDEN] = s * bh_ref[:]
    cat_ref[:, HIDDEN:] = s
    mu = jnp.mean(m, axis=-1, keepdims=True)
    var = jnp.mean((m - mu) ** 2, axis=-1, keepdims=True)
    h = (m - mu) * jax.lax.rsqrt(var + 1e-5) * g_ref[:] + be_ref[:]
    ynew_ref[:] = y_ref[:] + h * jax.nn.sigmoid(h)


def _edge_stage(xs, xd, ye, bh, y, g, be):
    e = xs.shape[0]
    spec = pl.BlockSpec((BM, HIDDEN), lambda i: (i, 0))
    vspec = pl.BlockSpec((1, HIDDEN), lambda i: (0, 0))
    cat, ynew = pl.pallas_call(
        _edge_body,
        grid=(e // BM,),
        in_specs=[spec, spec, spec, spec, spec, vspec, vspec],
        out_specs=[pl.BlockSpec((BM, 2 * HIDDEN), lambda i: (i, 0)), spec],
        out_shape=[
            jax.ShapeDtypeStruct((e, 2 * HIDDEN), jnp.float32),
            jax.ShapeDtypeStruct((e, HIDDEN), jnp.float32),
        ],
    )(xs, xd, ye, bh, y, g.reshape(1, HIDDEN), be.reshape(1, HIDDEN))
    return cat, ynew


def _node_body(xu_ref, ssh_ref, ss_ref, x_ref, g_ref, be_ref, o_ref):
    h = xu_ref[:] + ssh_ref[:] / (ss_ref[:] + 1e-6)
    mu = jnp.mean(h, axis=-1, keepdims=True)
    var = jnp.mean((h - mu) ** 2, axis=-1, keepdims=True)
    h = (h - mu) * jax.lax.rsqrt(var + 1e-5) * g_ref[:] + be_ref[:]
    o_ref[:] = x_ref[:] + h * jax.nn.sigmoid(h)


def _node_stage(xu, ssh, ss, x, g, be):
    n0 = x.shape[0]
    xu, ssh, ss, x = (_pad_rows(a, BM) for a in (xu, ssh, ss, x))
    n = x.shape[0]
    spec = pl.BlockSpec((BM, HIDDEN), lambda i: (i, 0))
    vspec = pl.BlockSpec((1, HIDDEN), lambda i: (0, 0))
    out = pl.pallas_call(
        _node_body,
        grid=(n // BM,),
        in_specs=[spec, spec, spec, spec, vspec, vspec],
        out_specs=spec,
        out_shape=jax.ShapeDtypeStruct((n, HIDDEN), jnp.float32),
    )(xu, ssh, ss, x, g.reshape(1, HIDDEN), be.reshape(1, HIDDEN))
    return out[:n0]


def _colsum_body(x_ref, o_ref):
    @pl.when(pl.program_id(0) == 0)
    def _init():
        o_ref[:] = jnp.zeros_like(o_ref)

    o_ref[:] += jnp.sum(x_ref[:], axis=0, keepdims=True)


def _colsum(x):
    xp = _pad_rows(x, BM)
    m = xp.shape[0]
    out = pl.pallas_call(
        _colsum_body,
        grid=(m // BM,),
        in_specs=[pl.BlockSpec((BM, HIDDEN), lambda i: (i, 0))],
        out_specs=pl.BlockSpec((1, HIDDEN), lambda i: (0, 0)),
        out_shape=jax.ShapeDtypeStruct((1, HIDDEN), jnp.float32),
    )(xp)
    return out[0]


def _egc(p, src, dst, x, y, n_nodes):
    wcat = jnp.concatenate(
        [p["src_gate"]["W"], p["dst_gate"]["W"], p["dst_update"]["W"],
         p["src_update"]["W"]], axis=1)
    bcat = jnp.concatenate(
        [p["src_gate"]["b"], p["dst_gate"]["b"], p["dst_update"]["b"],
         p["src_update"]["b"]], axis=0)
    xw = _linear(x, wcat, bcat)
    ye = _linear(y, p["edge_gate"]["W"], p["edge_gate"]["b"])
    xs = xw[:, :HIDDEN][src]
    xd = xw[:, HIDDEN:2 * HIDDEN][dst]
    bh = xw[:, 2 * HIDDEN:3 * HIDDEN][src]
    cat, y_out = _edge_stage(xs, xd, ye, bh, y, p["ln_e_g"], p["ln_e_b"])
    sums = jnp.zeros((n_nodes, 2 * HIDDEN), jnp.float32).at[dst].add(cat)
    x_out = _node_stage(xw[:, 3 * HIDDEN:], sums[:, :HIDDEN], sums[:, HIDDEN:],
                        x, p["ln_n_g"], p["ln_n_b"])
    return x_out, y_out


def _rbf(d, vmin, vmax, bins):
    centers = jnp.linspace(vmin, vmax, bins)
    gamma = 1.0 / ((vmax - vmin) / (bins - 1))
    return jnp.exp(-gamma * (d[:, None] - centers[None, :]) ** 2)


def kernel(atom_features, r, angle_h, params, edge_index, lg_edge_index):
    n = atom_features.shape[0]
    e = r.shape[0]
    src, dst = edge_index[0], edge_index[1]
    lsrc, ldst = lg_edge_index[0], lg_edge_index[1]

    z = _mlp(_mlp(_rbf(angle_h, -1.0, 1.0, 40), params["angle_emb"]["m1"]),
             params["angle_emb"]["m2"])
    x = _mlp(atom_features, params["atom_emb"])
    bondlength = jnp.linalg.norm(r, axis=1)
    y = _mlp(_mlp(_rbf(bondlength, 0.0, 8.0, 16), params["edge_emb"]["m1"]),
             params["edge_emb"]["m2"])

    for lp in params["alignn"]:
        x, m = _egc(lp["node"], src, dst, x, y, n)
        y, z = _egc(lp["edge"], lsrc, ldst, m, z, e)
    for gp in params["gcn"]:
        x, y = _egc(gp, src, dst, x, y, n)

    h = _colsum(x) / n
    out = h @ params["fc"]["W"] + params["fc"]["b"]
    return jnp.squeeze(out)


# speedup vs baseline: 1.1093x; 1.1093x over previous
import functools

import jax
import jax.numpy as jnp
from jax import lax
from jax.experimental import pallas as pl
from jax.experimental.pallas import tpu as pltpu
from jax.experimental.pallas import tpu_sc as plsc

HIDDEN = 256
BM = 640

_NC, _NS = 2, 16
_NW = _NC * _NS


def _sc_gather(table, idx, chunk):
    b = idx.shape[0]
    d = table.shape[1]
    n_chunks = b // chunk
    mesh = plsc.VectorSubcoreMesh(core_axis_name="c", subcore_axis_name="s")

    @functools.partial(
        pl.kernel,
        mesh=mesh,
        out_type=jax.ShapeDtypeStruct((b, d), jnp.float32),
        scratch_types=[
            pltpu.VMEM((chunk,), jnp.int32),
            pltpu.VMEM((chunk, d), jnp.float32),
            pltpu.SemaphoreType.DMA,
        ],
    )
    def k(table_hbm, idx_hbm, out_hbm, idx_v, rows_v, sem):
        wid = lax.axis_index("s") * _NC + lax.axis_index("c")
        n_mine = (n_chunks - wid + _NW - 1) // _NW

        def body(i, carry):
            c = wid + i * _NW
            off = c * chunk
            pltpu.sync_copy(idx_hbm.at[pl.ds(off, chunk)], idx_v)
            pltpu.async_copy(table_hbm.at[idx_v], rows_v, sem).wait()
            pltpu.sync_copy(rows_v, out_hbm.at[pl.ds(off, chunk)])
            return carry

        lax.fori_loop(0, n_mine, body, 0)

    return k(table, idx)


def _pad_rows(x, bm):
    pad = (-x.shape[0]) % bm
    if pad:
        x = jnp.pad(x, ((0, pad),) + ((0, 0),) * (x.ndim - 1))
    return x


def _lin_body(x_ref, w_ref, b_ref, o_ref):
    o_ref[:] = (
        jnp.dot(x_ref[:], w_ref[:], preferred_element_type=jnp.float32) + b_ref[:]
    )


def _lin_ln_silu_body(x_ref, w_ref, b_ref, g_ref, be_ref, o_ref):
    h = jnp.dot(x_ref[:], w_ref[:], preferred_element_type=jnp.float32) + b_ref[:]
    mu = jnp.mean(h, axis=-1, keepdims=True)
    var = jnp.mean((h - mu) ** 2, axis=-1, keepdims=True)
    h = (h - mu) * jax.lax.rsqrt(var + 1e-5) * g_ref[:] + be_ref[:]
    o_ref[:] = h * jax.nn.sigmoid(h)


def _linear(x, W, b, g=None, be=None):
    m0, k = x.shape
    f = W.shape[1]
    xp = _pad_rows(x, BM)
    m = xp.shape[0]
    args = [xp, W, b.reshape(1, f)]
    in_specs = [
        pl.BlockSpec((BM, k), lambda i: (i, 0)),
        pl.BlockSpec((k, f), lambda i: (0, 0)),
        pl.BlockSpec((1, f), lambda i: (0, 0)),
    ]
    if g is not None:
        args += [g.reshape(1, f), be.reshape(1, f)]
        in_specs += [
            pl.BlockSpec((1, f), lambda i: (0, 0)),
            pl.BlockSpec((1, f), lambda i: (0, 0)),
        ]
        body = _lin_ln_silu_body
    else:
        body = _lin_body
    out = pl.pallas_call(
        body,
        grid=(m // BM,),
        in_specs=in_specs,
        out_specs=pl.BlockSpec((BM, f), lambda i: (i, 0)),
        out_shape=jax.ShapeDtypeStruct((m, f), jnp.float32),
    )(*args)
    return out[:m0]


def _mlp(x, p):
    return _linear(x, p["W"], p["b"], p["g"], p["be"])


def _lin3_body(x_ref, w_ref, b_ref, o1_ref, o2_ref, o3_ref):
    h = jnp.dot(x_ref[:], w_ref[:], preferred_element_type=jnp.float32) + b_ref[:]
    d1 = o1_ref.shape[1]
    d2 = o2_ref.shape[1]
    o1_ref[:] = h[:, :d1]
    o2_ref[:] = h[:, d1:d1 + d2]
    o3_ref[:] = h[:, d1 + d2:]


def _linear3(x, W, b, splits):
    m0, k = x.shape
    f = W.shape[1]
    xp = _pad_rows(x, BM)
    m = xp.shape[0]
    outs = pl.pallas_call(
        _lin3_body,
        grid=(m // BM,),
        in_specs=[
            pl.BlockSpec((BM, k), lambda i: (i, 0)),
            pl.BlockSpec((k, f), lambda i: (0, 0)),
            pl.BlockSpec((1, f), lambda i: (0, 0)),
        ],
        out_specs=[pl.BlockSpec((BM, s), lambda i: (i, 0)) for s in splits],
        out_shape=[jax.ShapeDtypeStruct((m, s), jnp.float32) for s in splits],
    )(xp, W, b.reshape(1, f))
    return outs


def _edge_body(xsbh_ref, xd_ref, ye_ref, y_ref, g_ref, be_ref,
               cat_ref, ynew_ref):
    m = xsbh_ref[:, :HIDDEN] + xd_ref[:] + ye_ref[:]
    s = jax.nn.sigmoid(m)
    cat_ref[:, :HIDDEN] = s * xsbh_ref[:, HIDDEN:]
    cat_ref[:, HIDDEN:] = s
    mu = jnp.mean(m, axis=-1, keepdims=True)
    var = jnp.mean((m - mu) ** 2, axis=-1, keepdims=True)
    h = (m - mu) * jax.lax.rsqrt(var + 1e-5) * g_ref[:] + be_ref[:]
    ynew_ref[:] = y_ref[:] + h * jax.nn.sigmoid(h)


def _edge_stage(xsbh, xd, ye, y, g, be):
    e = xd.shape[0]
    spec = pl.BlockSpec((BM, HIDDEN), lambda i: (i, 0))
    spec2 = pl.BlockSpec((BM, 2 * HIDDEN), lambda i: (i, 0))
    vspec = pl.BlockSpec((1, HIDDEN), lambda i: (0, 0))
    cat, ynew = pl.pallas_call(
        _edge_body,
        grid=(e // BM,),
        in_specs=[spec2, spec, spec, spec, vspec, vspec],
        out_specs=[spec2, spec],
        out_shape=[
            jax.ShapeDtypeStruct((e, 2 * HIDDEN), jnp.float32),
            jax.ShapeDtypeStruct((e, HIDDEN), jnp.float32),
        ],
    )(xsbh, xd, ye, y, g.reshape(1, HIDDEN), be.reshape(1, HIDDEN))
    return cat, ynew


def _node_body(xu_ref, ssh_ref, ss_ref, x_ref, g_ref, be_ref, o_ref):
    h = xu_ref[:] + ssh_ref[:] / (ss_ref[:] + 1e-6)
    mu = jnp.mean(h, axis=-1, keepdims=True)
    var = jnp.mean((h - mu) ** 2, axis=-1, keepdims=True)
    h = (h - mu) * jax.lax.rsqrt(var + 1e-5) * g_ref[:] + be_ref[:]
    o_ref[:] = x_ref[:] + h * jax.nn.sigmoid(h)


def _node_stage(xu, ssh, ss, x, g, be):
    n0 = x.shape[0]
    xu, ssh, ss, x = (_pad_rows(a, BM) for a in (xu, ssh, ss, x))
    n = x.shape[0]
    spec = pl.BlockSpec((BM, HIDDEN), lambda i: (i, 0))
    vspec = pl.BlockSpec((1, HIDDEN), lambda i: (0, 0))
    out = pl.pallas_call(
        _node_body,
        grid=(n // BM,),
        in_specs=[spec, spec, spec, spec, vspec, vspec],
        out_specs=spec,
        out_shape=jax.ShapeDtypeStruct((n, HIDDEN), jnp.float32),
    )(xu, ssh, ss, x, g.reshape(1, HIDDEN), be.reshape(1, HIDDEN))
    return out[:n0]


def _colsum_body(x_ref, o_ref):
    @pl.when(pl.program_id(0) == 0)
    def _init():
        o_ref[:] = jnp.zeros_like(o_ref)

    o_ref[:] += jnp.sum(x_ref[:], axis=0, keepdims=True)


def _colsum(x):
    xp = _pad_rows(x, BM)
    m = xp.shape[0]
    out = pl.pallas_call(
        _colsum_body,
        grid=(m // BM,),
        in_specs=[pl.BlockSpec((BM, HIDDEN), lambda i: (i, 0))],
        out_specs=pl.BlockSpec((1, HIDDEN), lambda i: (0, 0)),
        out_shape=jax.ShapeDtypeStruct((1, HIDDEN), jnp.float32),
    )(xp)
    return out[0]


def _egc(p, src, dst, x, y, n_nodes):
    wcat = jnp.concatenate(
        [p["src_gate"]["W"], p["dst_update"]["W"], p["dst_gate"]["W"],
         p["src_update"]["W"]], axis=1)
    bcat = jnp.concatenate(
        [p["src_gate"]["b"], p["dst_update"]["b"], p["dst_gate"]["b"],
         p["src_update"]["b"]], axis=0)
    xsbh_t, xd_t, xu = _linear3(x, wcat, bcat, (2 * HIDDEN, HIDDEN, HIDDEN))
    ye = _linear(y, p["edge_gate"]["W"], p["edge_gate"]["b"])
    xsbh = _sc_gather(xsbh_t, src, 64)
    xd = _sc_gather(xd_t, dst, 128)
    cat, y_out = _edge_stage(xsbh, xd, ye, y, p["ln_e_g"], p["ln_e_b"])
    sums = jnp.zeros((n_nodes, 2 * HIDDEN), jnp.float32).at[dst].add(cat)
    x_out = _node_stage(xu[:x.shape[0]], sums[:, :HIDDEN], sums[:, HIDDEN:],
                        x, p["ln_n_g"], p["ln_n_b"])
    return x_out, y_out


def _rbf(d, vmin, vmax, bins):
    centers = jnp.linspace(vmin, vmax, bins)
    gamma = 1.0 / ((vmax - vmin) / (bins - 1))
    return jnp.exp(-gamma * (d[:, None] - centers[None, :]) ** 2)


def kernel(atom_features, r, angle_h, params, edge_index, lg_edge_index):
    n = atom_features.shape[0]
    e = r.shape[0]
    src, dst = edge_index[0], edge_index[1]
    lsrc, ldst = lg_edge_index[0], lg_edge_index[1]

    z = _mlp(_mlp(_rbf(angle_h, -1.0, 1.0, 40), params["angle_emb"]["m1"]),
             params["angle_emb"]["m2"])
    x = _mlp(atom_features, params["atom_emb"])
    bondlength = jnp.linalg.norm(r, axis=1)
    y = _mlp(_mlp(_rbf(bondlength, 0.0, 8.0, 16), params["edge_emb"]["m1"]),
             params["edge_emb"]["m2"])

    for lp in params["alignn"]:
        x, m = _egc(lp["node"], src, dst, x, y, n)
        y, z = _egc(lp["edge"], lsrc, ldst, m, z, e)
    for gp in params["gcn"]:
        x, y = _egc(gp, src, dst, x, y, n)

    h = _colsum(x) / n
    out = h @ params["fc"]["W"] + params["fc"]["b"]
    return jnp.squeeze(out)
